# Initial kernel scaffold; baseline (speedup 1.0000x reference)
import jax
import jax.numpy as jnp
from jax.experimental import pallas as pl


def kernel(p, active_nodes):
    raise NotImplementedError("write your pallas kernel here")


# speedup vs baseline: 6.7115x; 6.7115x over previous
import functools

import jax
import jax.numpy as jnp
from jax.experimental import pallas as pl
from jax.experimental.pallas import tpu as pltpu

_K = 16
_BIG_IDX = 2**30


def _knn_block_kernel(prow_ref, pcols_ref, out_ref, *, n, k):
    xi = prow_ref[:, 0:1]
    yi = prow_ref[:, 1:2]
    zi = prow_ref[:, 2:3]
    dx = xi - pcols_ref[0:1, :]
    dy = yi - pcols_ref[1:2, :]
    dz = zi - pcols_ref[2:3, :]
    d = dx * dx + dy * dy + dz * dz
    r = d.shape[0]
    iota = jax.lax.broadcasted_iota(jnp.int32, (r, n), 1)
    cols = []
    for _ in range(k):
        m = jnp.min(d, axis=1, keepdims=True)
        idx = jnp.min(jnp.where(d == m, iota, _BIG_IDX), axis=1,
                      keepdims=True)
        cols.append(idx)
        d = jnp.where(iota == idx, jnp.inf, d)
    out_ref[:, :] = jnp.concatenate(cols, axis=1)


@jax.jit
def kernel(p, active_nodes):
    n = p.shape[0]
    block_r = 256
    pcols = jnp.zeros((8, n), dtype=p.dtype).at[:3, :].set(p.T)
    idxs = pl.pallas_call(
        functools.partial(_knn_block_kernel, n=n, k=_K),
        grid=(n // block_r,),
        in_specs=[
            pl.BlockSpec((block_r, 3), lambda i: (i, 0)),
            pl.BlockSpec((8, n), lambda i: (0, 0)),
        ],
        out_specs=pl.BlockSpec((block_r, _K), lambda i: (i, 0)),
        out_shape=jax.ShapeDtypeStruct((n, _K), jnp.int32),
        compiler_params=pltpu.CompilerParams(
            dimension_semantics=("arbitrary",),
        ),
    )(p, pcols)
    row = jnp.broadcast_to(jnp.arange(n, dtype=idxs.dtype)[:, None], (n, _K))
    s = jnp.where(active_nodes[:, None], idxs, n - 1)
    r = jnp.where(active_nodes[:, None], row, n - 1)
    return s.reshape(-1), r.reshape(-1)


# speedup vs baseline: 7.1728x; 1.0687x over previous
import functools

import jax
import jax.numpy as jnp
from jax.experimental import pallas as pl
from jax.experimental.pallas import tpu as pltpu

_K = 16
_BIG_IDX = 2**30


def _knn_block_kernel(prow_ref, pcols_ref, out_ref, *, n, k):
    xi = prow_ref[:, 0:1]
    yi = prow_ref[:, 1:2]
    zi = prow_ref[:, 2:3]
    dx = xi - pcols_ref[0:1, :]
    dy = yi - pcols_ref[1:2, :]
    dz = zi - pcols_ref[2:3, :]
    d = dx * dx + dy * dy + dz * dz
    r = d.shape[0]
    iota = jax.lax.broadcasted_iota(jnp.int32, (r, n), 1)
    cols = []
    for _ in range(k):
        idx = jnp.argmin(d, axis=1).astype(jnp.int32)[:, None]
        cols.append(idx)
        d = jnp.where(iota == idx, jnp.inf, d)
    out_ref[:, :] = jnp.concatenate(cols, axis=1)


@jax.jit
def kernel(p, active_nodes):
    n = p.shape[0]
    block_r = 256
    pcols = jnp.zeros((8, n), dtype=p.dtype).at[:3, :].set(p.T)
    idxs = pl.pallas_call(
        functools.partial(_knn_block_kernel, n=n, k=_K),
        grid=(n // block_r,),
        in_specs=[
            pl.BlockSpec((block_r, 3), lambda i: (i, 0)),
            pl.BlockSpec((8, n), lambda i: (0, 0)),
        ],
        out_specs=pl.BlockSpec((block_r, _K), lambda i: (i, 0)),
        out_shape=jax.ShapeDtypeStruct((n, _K), jnp.int32),
        compiler_params=pltpu.CompilerParams(
            dimension_semantics=("arbitrary",),
        ),
    )(p, pcols)
    row = jnp.broadcast_to(jnp.arange(n, dtype=idxs.dtype)[:, None], (n, _K))
    s = jnp.where(active_nodes[:, None], idxs, n - 1)
    r = jnp.where(active_nodes[:, None], row, n - 1)
    return s.reshape(-1), r.reshape(-1)
